# baseline (speedup 1.0000x reference)
import jax
import jax.numpy as jnp
from jax.experimental import pallas as pl

_N = 10000
_H = 8
_D_OUT = 64


def _elu_body(x_ref, o_ref):
    x = x_ref[...]
    o_ref[...] = jnp.where(x > 0, x, jnp.exp(jnp.minimum(x, 0.0)) - 1.0)


def kernel(adj, x, W, attn_l, attn_r, bias):
    src = adj[0]
    dst = adj[1]
    n = x.shape[0]
    feat = (x @ W).reshape(n, _H, _D_OUT)
    el = jnp.sum(feat * attn_l[None, :, :], axis=-1)
    er = jnp.sum(feat * attn_r[None, :, :], axis=-1)
    e = el[src] + er[dst]
    e = jax.nn.leaky_relu(e, negative_slope=0.2)
    emax = jax.ops.segment_max(e, dst, num_segments=n)
    ex = jnp.exp(e - emax[dst])
    denom = jax.ops.segment_sum(ex, dst, num_segments=n)
    alpha = ex / denom[dst]
    msg = feat[src] * alpha[:, :, None]
    rst = jax.ops.segment_sum(msg, dst, num_segments=n)
    rst = rst + bias.reshape(1, _H, _D_OUT)
    h = rst.reshape(n, -1)
    return pl.pallas_call(
        _elu_body,
        out_shape=jax.ShapeDtypeStruct((n, _H * _D_OUT), jnp.float32),
        grid=(25,),
        in_specs=[pl.BlockSpec((400, 512), lambda i: (i, 0))],
        out_specs=pl.BlockSpec((400, 512), lambda i: (i, 0)),
    )(h)


# speedup vs baseline: 9.9514x; 9.9514x over previous
import jax
import jax.numpy as jnp
from jax import lax
from jax.experimental import pallas as pl
from jax.experimental.pallas import tpu as pltpu
from jax.experimental.pallas import tpu_sc as plsc

N = 10000
E = 320000
DIN = 128
H = 8
DO = 64
F = H * DO
FX = 640

NPAD = 10240
NS = 16
TROWS = 160
CH = 1600
NCHUNK = E // CH
G = 16

BA = 256
BD = 400


def _a_body(x_ref, w_ref, al_ref, ar_ref, feat_ref, er_ref, cm_ref):
    i = pl.program_id(0)
    fb = jnp.dot(x_ref[...], w_ref[...], preferred_element_type=jnp.float32)
    jidx = lax.broadcasted_iota(jnp.int32, (F, H), 0)
    hidx = lax.broadcasted_iota(jnp.int32, (F, H), 1)
    sel = jnp.where((jidx // DO) == hidx, 1.0, 0.0).astype(jnp.float32)
    elb = jnp.dot(fb * al_ref[...], sel, preferred_element_type=jnp.float32)
    erb = jnp.dot(fb * ar_ref[...], sel, preferred_element_type=jnp.float32)
    feat_ref[...] = jnp.concatenate(
        [fb, elb, jnp.zeros((BA, FX - F - H), jnp.float32)], axis=1)
    er_ref[...] = erb
    m = jnp.concatenate(
        [jnp.max(elb, axis=0, keepdims=True), jnp.max(erb, axis=0, keepdims=True)],
        axis=0)

    @pl.when(i == 0)
    def _():
        cm_ref[...] = m

    @pl.when(i > 0)
    def _():
        cm_ref[...] = jnp.maximum(cm_ref[...], m)


def _run_a(xp, W, alf, arf):
    return pl.pallas_call(
        _a_body,
        grid=(NPAD // BA,),
        in_specs=[
            pl.BlockSpec((BA, DIN), lambda i: (i, 0)),
            pl.BlockSpec((DIN, F), lambda i: (0, 0)),
            pl.BlockSpec((1, F), lambda i: (0, 0)),
            pl.BlockSpec((1, F), lambda i: (0, 0)),
        ],
        out_specs=[
            pl.BlockSpec((BA, FX), lambda i: (i, 0)),
            pl.BlockSpec((BA, H), lambda i: (i, 0)),
            pl.BlockSpec((2, H), lambda i: (0, 0)),
        ],
        out_shape=[
            jax.ShapeDtypeStruct((NPAD, FX), jnp.float32),
            jax.ShapeDtypeStruct((NPAD, H), jnp.float32),
            jax.ShapeDtypeStruct((2, H), jnp.float32),
        ],
    )(xp, W, alf, arf)


def _c_body(src_h, dst_h, feat_h, er_h, cm_h, acc_h, den_h,
            sstg, dstg, csrc, cdg, csrcg, cdgg,
            fbuf, erwin, exbuf, accf, denf, cmv, cvb):
    c = lax.axis_index("c")
    s = lax.axis_index("s")
    iota = lax.iota(jnp.int32, 16)
    i8 = iota % 8
    rpair = iota // 8
    zero16f = jnp.zeros((16,), jnp.float32)

    pltpu.sync_copy(cm_h, cmv)
    clv = plsc.load_gather(cmv, [iota * 0, i8])
    crv = plsc.load_gather(cmv, [iota * 0 + 1, i8])
    sm = clv + crv
    cvb[...] = jnp.maximum(sm, 0.2 * sm)

    for p in range(2):
        lo = p * 5120 + c * 2560 + s * TROWS
        hi = lo + TROWS

        def _zacc(j, carry):
            accf[pl.ds(j * 16, 16)] = zero16f
            return carry
        lax.fori_loop(0, TROWS * F // 16, _zacc, 0, unroll=8)

        def _zden(j, carry):
            denf[pl.ds(j * 16, 16)] = zero16f
            return carry
        lax.fori_loop(0, TROWS * H // 16, _zden, 0, unroll=8)

        pltpu.sync_copy(er_h.at[pl.ds(lo, TROWS)], erwin)

        def _flush(cnt):
            for j in range(G // 16):
                idxv = j * 16 + iota
                mv = idxv < cnt
                csrcg[pl.ds(j * 16, 16)] = jnp.where(mv, csrc[pl.ds(j * 16, 16)], 0)
                cdgg[pl.ds(j * 16, 16)] = jnp.where(mv, cdg[pl.ds(j * 16, 16)], lo)
            pltpu.sync_copy(feat_h.at[csrcg], fbuf)
            cv = cvb[...]
            for j in range(G * H // 16):
                erow = 2 * j + rpair
                elv = plsc.load_gather(fbuf, [erow, F + i8])
                dl2 = plsc.load_gather(cdgg, [erow]) - lo
                erv = plsc.load_gather(erwin, [dl2, i8])
                z = elv + erv
                z = jnp.maximum(z, 0.2 * z)
                exv = jnp.exp(z - cv)
                exv = jnp.where(erow < cnt, exv, 0.0)
                plsc.store_scatter(exbuf, [erow, i8], exv)
                plsc.addupdate_scatter(denf, [dl2 * H + i8], exv)

            def _acc1(e, carry):
                dv16 = plsc.load_gather(cdgg, [iota * 0 + e])
                base = (dv16[0] - lo) * F
                for h_ in range(H):
                    scv = plsc.load_gather(exbuf, [iota * 0 + e, iota * 0 + h_])
                    for v_ in range(DO // 16):
                        off = h_ * DO + v_ * 16
                        fm = fbuf[e, pl.ds(off, 16)] * scv
                        plsc.addupdate(accf.at[pl.ds(base + off, 16)], fm)
                return carry
            lax.fori_loop(0, G, _acc1, 0)

        def _chunk(k, pos):
            base = k * CH
            pltpu.sync_copy(src_h.at[pl.ds(base, CH)], sstg)
            pltpu.sync_copy(dst_h.at[pl.ds(base, CH)], dstg)

            def _vec(i2, pos_):
                dv = dstg[pl.ds(i2 * 16, 16)]
                sv = sstg[pl.ds(i2 * 16, 16)]
                m = (dv >= lo) & (dv < hi)
                plsc.store_compressed(csrc.at[pl.ds(pos_, 16)], sv, mask=m)
                plsc.store_compressed(cdg.at[pl.ds(pos_, 16)], dv, mask=m)
                pos2 = pos_ + jnp.sum(jnp.where(m, 1, 0))

                def _full(pv):
                    _flush(G)
                    lv = csrc[pl.ds(G, 16)]
                    csrc[pl.ds(0, 16)] = lv
                    lv2 = cdg[pl.ds(G, 16)]
                    cdg[pl.ds(0, 16)] = lv2
                    return pv - G

                return lax.cond(pos2 >= G, _full, lambda pv: pv, pos2)

            return lax.fori_loop(0, CH // 16, _vec, pos)

        pos = lax.fori_loop(0, NCHUNK, _chunk, jnp.int32(0))

        @pl.when(pos > 0)
        def _():
            _flush(pos)

        pltpu.sync_copy(accf, acc_h.at[pl.ds(lo * F, TROWS * F)])
        pltpu.sync_copy(denf, den_h.at[pl.ds(lo * H, TROWS * H)])


def _run_c(src, dst, feat, er, cmax):
    kc = pl.kernel(
        _c_body,
        out_type=(jax.ShapeDtypeStruct((NPAD * F,), jnp.float32),
                  jax.ShapeDtypeStruct((NPAD * H,), jnp.float32)),
        mesh=plsc.VectorSubcoreMesh(core_axis_name="c", subcore_axis_name="s"),
        compiler_params=pltpu.CompilerParams(needs_layout_passes=False),
        scratch_types=[
            pltpu.VMEM((CH,), jnp.int32),
            pltpu.VMEM((CH,), jnp.int32),
            pltpu.VMEM((G + 16,), jnp.int32),
            pltpu.VMEM((G + 16,), jnp.int32),
            pltpu.VMEM((G,), jnp.int32),
            pltpu.VMEM((G,), jnp.int32),
            pltpu.VMEM((G, FX), jnp.float32),
            pltpu.VMEM((TROWS, H), jnp.float32),
            pltpu.VMEM((G, H), jnp.float32),
            pltpu.VMEM((TROWS * F,), jnp.float32),
            pltpu.VMEM((TROWS * H,), jnp.float32),
            pltpu.VMEM((2, H), jnp.float32),
            pltpu.VMEM((16,), jnp.float32),
        ],
    )
    return kc(src, dst, feat, er, cmax)


def _d_body(acc_ref, den_ref, b_ref, o_ref):
    d = den_ref[...]
    dsafe = jnp.where(d > 0, d, 1.0)
    rinv = 1.0 / dsafe
    hidx = lax.broadcasted_iota(jnp.int32, (H, F), 0)
    jidx = lax.broadcasted_iota(jnp.int32, (H, F), 1)
    sel = jnp.where(hidx == (jidx // DO), 1.0, 0.0).astype(jnp.float32)
    rbig = jnp.dot(rinv, sel, preferred_element_type=jnp.float32)
    y = acc_ref[...] * rbig + b_ref[...]
    o_ref[...] = jnp.where(y > 0, y, jnp.exp(jnp.minimum(y, 0.0)) - 1.0)


def _run_d(acc, den, biasr):
    return pl.pallas_call(
        _d_body,
        grid=(N // BD,),
        in_specs=[
            pl.BlockSpec((BD, F), lambda i: (i, 0)),
            pl.BlockSpec((BD, H), lambda i: (i, 0)),
            pl.BlockSpec((1, F), lambda i: (0, 0)),
        ],
        out_specs=pl.BlockSpec((BD, F), lambda i: (i, 0)),
        out_shape=jax.ShapeDtypeStruct((N, F), jnp.float32),
    )(acc, den, biasr)


def kernel(adj, x, W, attn_l, attn_r, bias):
    src = adj[0].astype(jnp.int32)
    dst = adj[1].astype(jnp.int32)
    xp = jnp.pad(x, ((0, NPAD - N), (0, 0)))
    alf = attn_l.reshape(1, F)
    arf = attn_r.reshape(1, F)
    feat, er, cmax = _run_a(xp, W, alf, arf)
    accflat, denflat = _run_c(src, dst, feat, er, cmax)
    acc = accflat.reshape(NPAD, F)
    den = denflat.reshape(NPAD, H)
    return _run_d(acc, den, bias.reshape(1, F))


# speedup vs baseline: 13.4136x; 1.3479x over previous
import jax
import jax.numpy as jnp
from jax import lax
from jax.experimental import pallas as pl
from jax.experimental.pallas import tpu as pltpu
from jax.experimental.pallas import tpu_sc as plsc

N = 10000
E = 320000
DIN = 128
H = 8
DO = 64
F = H * DO
FX = 640

NPAD = 10240
NS = 16
TROWS = 160
CH = 800
NCHUNK = E // CH
G = 16

BA = 256
BD = 400


def _a_body(x_ref, w_ref, al_ref, ar_ref, feat_ref, er_ref, cm_ref):
    i = pl.program_id(0)
    fb = jnp.dot(x_ref[...], w_ref[...], preferred_element_type=jnp.float32)
    jidx = lax.broadcasted_iota(jnp.int32, (F, H), 0)
    hidx = lax.broadcasted_iota(jnp.int32, (F, H), 1)
    sel = jnp.where((jidx // DO) == hidx, 1.0, 0.0).astype(jnp.float32)
    elb = jnp.dot(fb * al_ref[...], sel, preferred_element_type=jnp.float32)
    erb = jnp.dot(fb * ar_ref[...], sel, preferred_element_type=jnp.float32)
    feat_ref[...] = jnp.concatenate(
        [fb, elb, jnp.zeros((BA, FX - F - H), jnp.float32)], axis=1)
    er_ref[...] = erb
    m = jnp.concatenate(
        [jnp.max(elb, axis=0, keepdims=True), jnp.max(erb, axis=0, keepdims=True)],
        axis=0)

    @pl.when(i == 0)
    def _():
        cm_ref[...] = m

    @pl.when(i > 0)
    def _():
        cm_ref[...] = jnp.maximum(cm_ref[...], m)


def _run_a(xp, W, alf, arf):
    return pl.pallas_call(
        _a_body,
        grid=(NPAD // BA,),
        in_specs=[
            pl.BlockSpec((BA, DIN), lambda i: (i, 0)),
            pl.BlockSpec((DIN, F), lambda i: (0, 0)),
            pl.BlockSpec((1, F), lambda i: (0, 0)),
            pl.BlockSpec((1, F), lambda i: (0, 0)),
        ],
        out_specs=[
            pl.BlockSpec((BA, FX), lambda i: (i, 0)),
            pl.BlockSpec((BA, H), lambda i: (i, 0)),
            pl.BlockSpec((2, H), lambda i: (0, 0)),
        ],
        out_shape=[
            jax.ShapeDtypeStruct((NPAD, FX), jnp.float32),
            jax.ShapeDtypeStruct((NPAD, H), jnp.float32),
            jax.ShapeDtypeStruct((2, H), jnp.float32),
        ],
    )(xp, W, alf, arf)


def _c_body(src_h, dst_h, feat_h, er_h, cm_h, acc_h, den_h,
            sstga, dstga, sstgb, dstgb, csrc, cdg, csrcg, cdgg,
            fbuf, erwin, exbuf, accf, denf, cmv, cvb, sema, semb, semf):
    c = lax.axis_index("c")
    s = lax.axis_index("s")
    iota = lax.iota(jnp.int32, 16)
    i8 = iota % 8
    rpair = iota // 8
    zero16f = jnp.zeros((16,), jnp.float32)

    pltpu.sync_copy(cm_h, cmv)
    clv = plsc.load_gather(cmv, [iota * 0, i8])
    crv = plsc.load_gather(cmv, [iota * 0 + 1, i8])
    sm = clv + crv
    cvb[...] = jnp.maximum(sm, 0.2 * sm)

    def _wait_stage(sbuf, dbuf, sem):
        pltpu.make_async_copy(src_h.at[pl.ds(0, CH)], sbuf, sem).wait()
        pltpu.make_async_copy(dst_h.at[pl.ds(0, CH)], dbuf, sem).wait()

    def _fire_stage(base, sbuf, dbuf, sem):
        pltpu.async_copy(src_h.at[pl.ds(base, CH)], sbuf, sem)
        pltpu.async_copy(dst_h.at[pl.ds(base, CH)], dbuf, sem)

    def _wait_gather():
        pltpu.make_async_copy(feat_h.at[csrcg], fbuf, semf).wait()

    for p in range(2):
        lo = p * 5120 + c * 2560 + s * TROWS
        hi = lo + TROWS

        def _zacc(j, carry):
            accf[pl.ds(j * 16, 16)] = zero16f
            return carry
        lax.fori_loop(0, TROWS * F // 16, _zacc, 0, unroll=8)

        def _zden(j, carry):
            denf[pl.ds(j * 16, 16)] = zero16f
            return carry
        lax.fori_loop(0, TROWS * H // 16, _zden, 0, unroll=8)

        pltpu.sync_copy(er_h.at[pl.ds(lo, TROWS)], erwin)

        def _build(cnt):
            for j in range(G // 16):
                idxv = j * 16 + iota
                mv = idxv < cnt
                csrcg[pl.ds(j * 16, 16)] = jnp.where(mv, csrc[pl.ds(j * 16, 16)], 0)
                cdgg[pl.ds(j * 16, 16)] = jnp.where(mv, cdg[pl.ds(j * 16, 16)], lo)

        def _process(cnt):
            cv = cvb[...]
            for j in range(G * H // 16):
                erow = 2 * j + rpair
                elv = plsc.load_gather(fbuf, [erow, F + i8])
                dl2 = plsc.load_gather(cdgg, [erow]) - lo
                erv = plsc.load_gather(erwin, [dl2, i8])
                z = elv + erv
                z = jnp.maximum(z, 0.2 * z)
                exv = jnp.exp(z - cv)
                exv = jnp.where(erow < cnt, exv, 0.0)
                plsc.store_scatter(exbuf, [erow, i8], exv)
                plsc.addupdate_scatter(denf, [dl2 * H + i8], exv)

            def _acc1(e, carry):
                dv16 = plsc.load_gather(cdgg, [iota * 0 + e])
                base = (dv16[0] - lo) * F
                for h_ in range(H):
                    scv = plsc.load_gather(exbuf, [iota * 0 + e, iota * 0 + h_])
                    for v_ in range(DO // 16):
                        off = h_ * DO + v_ * 16
                        fm = fbuf[e, pl.ds(off, 16)] * scv
                        plsc.addupdate(accf.at[pl.ds(base + off, 16)], fm)
                return carry
            lax.fori_loop(0, G, _acc1, 0)

        _build(jnp.int32(0))
        pltpu.async_copy(feat_h.at[csrcg], fbuf, semf)

        def _scan_buf(sbuf, dbuf, st):
            def _vec(i2, st_):
                pos_, pcnt_ = st_
                dv = dbuf[pl.ds(i2 * 16, 16)]
                sv = sbuf[pl.ds(i2 * 16, 16)]
                m = (dv >= lo) & (dv < hi)
                plsc.store_compressed(csrc.at[pl.ds(pos_, 16)], sv, mask=m)
                plsc.store_compressed(cdg.at[pl.ds(pos_, 16)], dv, mask=m)
                pos2 = pos_ + jnp.sum(jnp.where(m, 1, 0))

                def _full(st2):
                    pv, pc = st2
                    _wait_gather()
                    _process(pc)
                    _build(jnp.int32(G))
                    pltpu.async_copy(feat_h.at[csrcg], fbuf, semf)
                    lv = csrc[pl.ds(G, 16)]
                    csrc[pl.ds(0, 16)] = lv
                    lv2 = cdg[pl.ds(G, 16)]
                    cdg[pl.ds(0, 16)] = lv2
                    return (pv - G, jnp.int32(G))

                return lax.cond(pos2 >= G, _full, lambda s2: s2, (pos2, pcnt_))

            return lax.fori_loop(0, CH // 16, _vec, st)

        _fire_stage(0, sstga, dstga, sema)

        def _pair(k, st):
            _fire_stage((2 * k + 1) * CH, sstgb, dstgb, semb)
            _wait_stage(sstga, dstga, sema)
            st = _scan_buf(sstga, dstga, st)
            nxt = jnp.minimum((2 * k + 2) * CH, E - CH)
            _fire_stage(nxt, sstga, dstga, sema)
            _wait_stage(sstgb, dstgb, semb)
            st = _scan_buf(sstgb, dstgb, st)
            return st

        pos, pcnt = lax.fori_loop(0, NCHUNK // 2, _pair,
                                  (jnp.int32(0), jnp.int32(0)))
        _wait_stage(sstga, dstga, sema)

        _wait_gather()
        _process(pcnt)

        @pl.when(pos > 0)
        def _():
            _build(pos)
            pltpu.async_copy(feat_h.at[csrcg], fbuf, semf)
            _wait_gather()
            _process(pos)

        pltpu.sync_copy(accf, acc_h.at[pl.ds(lo * F, TROWS * F)])
        pltpu.sync_copy(denf, den_h.at[pl.ds(lo * H, TROWS * H)])


def _run_c(src, dst, feat, er, cmax):
    kc = pl.kernel(
        _c_body,
        out_type=(jax.ShapeDtypeStruct((NPAD * F,), jnp.float32),
                  jax.ShapeDtypeStruct((NPAD * H,), jnp.float32)),
        mesh=plsc.VectorSubcoreMesh(core_axis_name="c", subcore_axis_name="s"),
        compiler_params=pltpu.CompilerParams(needs_layout_passes=False),
        scratch_types=[
            pltpu.VMEM((CH,), jnp.int32),
            pltpu.VMEM((CH,), jnp.int32),
            pltpu.VMEM((CH,), jnp.int32),
            pltpu.VMEM((CH,), jnp.int32),
            pltpu.VMEM((G + 16,), jnp.int32),
            pltpu.VMEM((G + 16,), jnp.int32),
            pltpu.VMEM((G,), jnp.int32),
            pltpu.VMEM((G,), jnp.int32),
            pltpu.VMEM((G, FX), jnp.float32),
            pltpu.VMEM((TROWS, H), jnp.float32),
            pltpu.VMEM((G, H), jnp.float32),
            pltpu.VMEM((TROWS * F,), jnp.float32),
            pltpu.VMEM((TROWS * H,), jnp.float32),
            pltpu.VMEM((2, H), jnp.float32),
            pltpu.VMEM((16,), jnp.float32),
            pltpu.SemaphoreType.DMA,
            pltpu.SemaphoreType.DMA,
            pltpu.SemaphoreType.DMA,
        ],
    )
    return kc(src, dst, feat, er, cmax)


def _d_body(acc_ref, den_ref, b_ref, o_ref):
    d = den_ref[...]
    dsafe = jnp.where(d > 0, d, 1.0)
    rinv = 1.0 / dsafe
    hidx = lax.broadcasted_iota(jnp.int32, (H, F), 0)
    jidx = lax.broadcasted_iota(jnp.int32, (H, F), 1)
    sel = jnp.where(hidx == (jidx // DO), 1.0, 0.0).astype(jnp.float32)
    rbig = jnp.dot(rinv, sel, preferred_element_type=jnp.float32)
    y = acc_ref[...] * rbig + b_ref[...]
    o_ref[...] = jnp.where(y > 0, y, jnp.exp(jnp.minimum(y, 0.0)) - 1.0)


def _run_d(acc, den, biasr):
    return pl.pallas_call(
        _d_body,
        grid=(N // BD,),
        in_specs=[
            pl.BlockSpec((BD, F), lambda i: (i, 0)),
            pl.BlockSpec((BD, H), lambda i: (i, 0)),
            pl.BlockSpec((1, F), lambda i: (0, 0)),
        ],
        out_specs=pl.BlockSpec((BD, F), lambda i: (i, 0)),
        out_shape=jax.ShapeDtypeStruct((N, F), jnp.float32),
    )(acc, den, biasr)


def kernel(adj, x, W, attn_l, attn_r, bias):
    src = adj[0].astype(jnp.int32)
    dst = adj[1].astype(jnp.int32)
    xp = jnp.pad(x, ((0, NPAD - N), (0, 0)))
    alf = attn_l.reshape(1, F)
    arf = attn_r.reshape(1, F)
    feat, er, cmax = _run_a(xp, W, alf, arf)
    accflat, denflat = _run_c(src, dst, feat, er, cmax)
    acc = accflat.reshape(NPAD, F)
    den = denflat.reshape(NPAD, H)
    return _run_d(acc, den, bias.reshape(1, F))


# speedup vs baseline: 15.4513x; 1.1519x over previous
import jax
import jax.numpy as jnp
from jax import lax
from jax.experimental import pallas as pl
from jax.experimental.pallas import tpu as pltpu
from jax.experimental.pallas import tpu_sc as plsc

N = 10000
E = 320000
DIN = 128
H = 8
DO = 64
F = H * DO
FX = 640

NPAD = 10240
NS = 16
TROWS = 160
CH = 1600
NCHUNK = E // CH
G = 16

BA = 256
BD = 400


def _a_body(x_ref, w_ref, al_ref, ar_ref, feat_ref, er_ref, cm_ref):
    i = pl.program_id(0)
    fb = jnp.dot(x_ref[...], w_ref[...], preferred_element_type=jnp.float32)
    jidx = lax.broadcasted_iota(jnp.int32, (F, H), 0)
    hidx = lax.broadcasted_iota(jnp.int32, (F, H), 1)
    sel = jnp.where((jidx // DO) == hidx, 1.0, 0.0).astype(jnp.float32)
    elb = jnp.dot(fb * al_ref[...], sel, preferred_element_type=jnp.float32)
    erb = jnp.dot(fb * ar_ref[...], sel, preferred_element_type=jnp.float32)
    feat_ref[...] = jnp.concatenate(
        [fb, elb, jnp.zeros((BA, FX - F - H), jnp.float32)], axis=1)
    er_ref[...] = erb
    m = jnp.concatenate(
        [jnp.max(elb, axis=0, keepdims=True), jnp.max(erb, axis=0, keepdims=True)],
        axis=0)

    @pl.when(i == 0)
    def _():
        cm_ref[...] = m

    @pl.when(i > 0)
    def _():
        cm_ref[...] = jnp.maximum(cm_ref[...], m)


def _run_a(xp, W, alf, arf):
    return pl.pallas_call(
        _a_body,
        grid=(NPAD // BA,),
        in_specs=[
            pl.BlockSpec((BA, DIN), lambda i: (i, 0)),
            pl.BlockSpec((DIN, F), lambda i: (0, 0)),
            pl.BlockSpec((1, F), lambda i: (0, 0)),
            pl.BlockSpec((1, F), lambda i: (0, 0)),
        ],
        out_specs=[
            pl.BlockSpec((BA, FX), lambda i: (i, 0)),
            pl.BlockSpec((BA, H), lambda i: (i, 0)),
            pl.BlockSpec((2, H), lambda i: (0, 0)),
        ],
        out_shape=[
            jax.ShapeDtypeStruct((NPAD, FX), jnp.float32),
            jax.ShapeDtypeStruct((NPAD, H), jnp.float32),
            jax.ShapeDtypeStruct((2, H), jnp.float32),
        ],
    )(xp, W, alf, arf)


def _c_body(src_h, dst_h, feat_h, er_h, cm_h, acc_h, den_h,
            sstga, dstga, sstgb, dstgb, cdg, csrcg, cdgg,
            fbuf, erwin, exbuf, accf, denf, cmv, cvb, sema, semb, semf):
    c = lax.axis_index("c")
    s = lax.axis_index("s")
    iota = lax.iota(jnp.int32, 16)
    i8 = iota % 8
    rpair = iota // 8
    zero16f = jnp.zeros((16,), jnp.float32)

    pltpu.sync_copy(cm_h, cmv)
    clv = plsc.load_gather(cmv, [iota * 0, i8])
    crv = plsc.load_gather(cmv, [iota * 0 + 1, i8])
    sm = clv + crv
    cvb[...] = jnp.maximum(sm, 0.2 * sm)

    def _wait_stage(sbuf, dbuf, sem):
        pltpu.make_async_copy(src_h.at[pl.ds(0, CH)], sbuf, sem).wait()
        pltpu.make_async_copy(dst_h.at[pl.ds(0, CH)], dbuf, sem).wait()

    def _fire_stage(base, sbuf, dbuf, sem):
        pltpu.async_copy(src_h.at[pl.ds(base, CH)], sbuf, sem)
        pltpu.async_copy(dst_h.at[pl.ds(base, CH)], dbuf, sem)

    def _wait_gather():
        pltpu.make_async_copy(feat_h.at[csrcg], fbuf, semf).wait()

    for p in range(2):
        lo = p * 5120 + c * 2560 + s * TROWS
        hi = lo + TROWS

        def _zacc(j, carry):
            accf[pl.ds(j * 16, 16)] = zero16f
            return carry
        lax.fori_loop(0, TROWS * F // 16, _zacc, 0, unroll=8)

        def _zden(j, carry):
            denf[pl.ds(j * 16, 16)] = zero16f
            return carry
        lax.fori_loop(0, TROWS * H // 16, _zden, 0, unroll=8)

        pltpu.sync_copy(er_h.at[pl.ds(lo, TROWS)], erwin)

        def _build(cnt):
            for j in range(G // 16):
                idxv = j * 16 + iota
                mv = idxv < cnt
                pk = cdg[pl.ds(j * 16, 16)]
                csrcg[pl.ds(j * 16, 16)] = jnp.where(mv, pk & 16383, 0)
                cdgg[pl.ds(j * 16, 16)] = jnp.where(mv, pk >> 14, lo)

        def _process(cnt):
            cv = cvb[...]
            for j in range(G * H // 16):
                erow = 2 * j + rpair
                elv = plsc.load_gather(fbuf, [erow, F + i8])
                dl2 = plsc.load_gather(cdgg, [erow]) - lo
                erv = plsc.load_gather(erwin, [dl2, i8])
                z = elv + erv
                z = jnp.maximum(z, 0.2 * z)
                exv = jnp.exp(z - cv)
                exv = jnp.where(erow < cnt, exv, 0.0)
                plsc.store_scatter(exbuf, [erow, i8], exv)
                plsc.addupdate_scatter(denf, [dl2 * H + i8], exv)

            def _acc1(e, carry):
                dv16 = plsc.load_gather(cdgg, [iota * 0 + e])
                base = (dv16[0] - lo) * F
                for h_ in range(H):
                    scv = plsc.load_gather(exbuf, [iota * 0 + e, iota * 0 + h_])
                    for v_ in range(DO // 16):
                        off = h_ * DO + v_ * 16
                        fm = fbuf[e, pl.ds(off, 16)] * scv
                        plsc.addupdate(accf.at[pl.ds(base + off, 16)], fm)
                return carry
            lax.fori_loop(0, G, _acc1, 0)

        _build(jnp.int32(0))
        pltpu.async_copy(feat_h.at[csrcg], fbuf, semf)

        def _full(st2):
            pv, pc = st2
            _wait_gather()
            _process(pc)
            _build(jnp.int32(G))
            pltpu.async_copy(feat_h.at[csrcg], fbuf, semf)
            lv = cdg[pl.ds(G, 16)]
            cdg[pl.ds(0, 16)] = lv
            lv2 = cdg[pl.ds(G + 16, 16)]
            cdg[pl.ds(16, 16)] = lv2
            return (pv - G, jnp.int32(G))

        def _scan_buf(sbuf, dbuf, st):
            def _vec(i2, st_):
                pos_, pcnt_ = st_
                dv0 = dbuf[pl.ds(i2 * 32, 16)]
                sv0 = sbuf[pl.ds(i2 * 32, 16)]
                dv1 = dbuf[pl.ds(i2 * 32 + 16, 16)]
                sv1 = sbuf[pl.ds(i2 * 32 + 16, 16)]
                m0 = (dv0 >= lo) & (dv0 < hi)
                m1 = (dv1 >= lo) & (dv1 < hi)
                cnt0 = jnp.sum(jnp.where(m0, 1, 0))
                cnt1 = jnp.sum(jnp.where(m1, 1, 0))
                pk0 = sv0 | (dv0 << 14)
                pk1 = sv1 | (dv1 << 14)
                plsc.store_compressed(cdg.at[pl.ds(pos_, 16)], pk0, mask=m0)
                pos1 = pos_ + cnt0
                plsc.store_compressed(cdg.at[pl.ds(pos1, 16)], pk1, mask=m1)
                pos2 = pos1 + cnt1

                st2 = lax.cond(pos2 >= G, _full, lambda s2: s2, (pos2, pcnt_))
                return lax.cond(st2[0] >= G, _full, lambda s2: s2, st2)

            return lax.fori_loop(0, CH // 32, _vec, st)

        _fire_stage(0, sstga, dstga, sema)

        def _pair(k, st):
            _fire_stage((2 * k + 1) * CH, sstgb, dstgb, semb)
            _wait_stage(sstga, dstga, sema)
            st = _scan_buf(sstga, dstga, st)
            nxt = jnp.minimum((2 * k + 2) * CH, E - CH)
            _fire_stage(nxt, sstga, dstga, sema)
            _wait_stage(sstgb, dstgb, semb)
            st = _scan_buf(sstgb, dstgb, st)
            return st

        pos, pcnt = lax.fori_loop(0, NCHUNK // 2, _pair,
                                  (jnp.int32(0), jnp.int32(0)))
        _wait_stage(sstga, dstga, sema)

        _wait_gather()
        _process(pcnt)

        @pl.when(pos > 0)
        def _():
            _build(pos)
            pltpu.async_copy(feat_h.at[csrcg], fbuf, semf)
            _wait_gather()
            _process(pos)

        pltpu.sync_copy(accf, acc_h.at[pl.ds(lo * F, TROWS * F)])
        pltpu.sync_copy(denf, den_h.at[pl.ds(lo * H, TROWS * H)])


def _run_c(src, dst, feat, er, cmax):
    kc = pl.kernel(
        _c_body,
        out_type=(jax.ShapeDtypeStruct((NPAD * F,), jnp.float32),
                  jax.ShapeDtypeStruct((NPAD * H,), jnp.float32)),
        mesh=plsc.VectorSubcoreMesh(core_axis_name="c", subcore_axis_name="s"),
        compiler_params=pltpu.CompilerParams(needs_layout_passes=False),
        scratch_types=[
            pltpu.VMEM((CH,), jnp.int32),
            pltpu.VMEM((CH,), jnp.int32),
            pltpu.VMEM((CH,), jnp.int32),
            pltpu.VMEM((CH,), jnp.int32),
            pltpu.VMEM((G + 48,), jnp.int32),
            pltpu.VMEM((G,), jnp.int32),
            pltpu.VMEM((G,), jnp.int32),
            pltpu.VMEM((G, FX), jnp.float32),
            pltpu.VMEM((TROWS, H), jnp.float32),
            pltpu.VMEM((G, H), jnp.float32),
            pltpu.VMEM((TROWS * F,), jnp.float32),
            pltpu.VMEM((TROWS * H,), jnp.float32),
            pltpu.VMEM((2, H), jnp.float32),
            pltpu.VMEM((16,), jnp.float32),
            pltpu.SemaphoreType.DMA,
            pltpu.SemaphoreType.DMA,
            pltpu.SemaphoreType.DMA,
        ],
    )
    return kc(src, dst, feat, er, cmax)


def _d_body(acc_ref, den_ref, b_ref, o_ref):
    d = den_ref[...]
    dsafe = jnp.where(d > 0, d, 1.0)
    rinv = 1.0 / dsafe
    hidx = lax.broadcasted_iota(jnp.int32, (H, F), 0)
    jidx = lax.broadcasted_iota(jnp.int32, (H, F), 1)
    sel = jnp.where(hidx == (jidx // DO), 1.0, 0.0).astype(jnp.float32)
    rbig = jnp.dot(rinv, sel, preferred_element_type=jnp.float32)
    y = acc_ref[...] * rbig + b_ref[...]
    o_ref[...] = jnp.where(y > 0, y, jnp.exp(jnp.minimum(y, 0.0)) - 1.0)


def _run_d(acc, den, biasr):
    return pl.pallas_call(
        _d_body,
        grid=(N // BD,),
        in_specs=[
            pl.BlockSpec((BD, F), lambda i: (i, 0)),
            pl.BlockSpec((BD, H), lambda i: (i, 0)),
            pl.BlockSpec((1, F), lambda i: (0, 0)),
        ],
        out_specs=pl.BlockSpec((BD, F), lambda i: (i, 0)),
        out_shape=jax.ShapeDtypeStruct((N, F), jnp.float32),
    )(acc, den, biasr)


def kernel(adj, x, W, attn_l, attn_r, bias):
    src = adj[0].astype(jnp.int32)
    dst = adj[1].astype(jnp.int32)
    xp = jnp.pad(x, ((0, NPAD - N), (0, 0)))
    alf = attn_l.reshape(1, F)
    arf = attn_r.reshape(1, F)
    feat, er, cmax = _run_a(xp, W, alf, arf)
    accflat, denflat = _run_c(src, dst, feat, er, cmax)
    acc = accflat.reshape(NPAD, F)
    den = denflat.reshape(NPAD, H)
    return _run_d(acc, den, bias.reshape(1, F))
